# Initial kernel scaffold; baseline (speedup 1.0000x reference)
import jax
import jax.numpy as jnp
from jax.experimental import pallas as pl


def kernel(features, edge_indexes_1, edge_indexes_3, edge_indexes_9, W1, b1, W2, b2, W3, b3):
    raise NotImplementedError("write your pallas kernel here")


# speedup vs baseline: 17.4314x; 17.4314x over previous
import functools

import jax
import jax.numpy as jnp
from jax import lax
from jax.experimental import pallas as pl
from jax.experimental.pallas import tpu as pltpu
from jax.experimental.pallas import tpu_sc as plsc

N = 10000
E = 320000
NC = 2
NS = 16
NW = NC * NS
EPW = E // NW
CH = 80
NCHUNK = EPW // CH
RPS = 624
TAIL = N - NS * RPS
DEG_W = 8

_MESH = plsc.VectorSubcoreMesh(core_axis_name="c", subcore_axis_name="s")
_SC_PARAMS = pltpu.CompilerParams(use_tc_tiling_on_sc=False)


def _striped(s, mk):
    mk(s * RPS, RPS)

    @pl.when(s == NS - 1)
    def _():
        mk(NS * RPS, TAIL)


def _deg_body(c1_hbm, c2_hbm, c3_hbm, ones_hbm, zeros_hbm, out_hbm,
              ones_v, idx_v, acc0, acc1, acc2):
    c = lax.axis_index("c")
    s = lax.axis_index("s")
    wid = c * NS + s
    for acc in (acc0, acc1, acc2):
        _striped(s, lambda o, n, acc=acc: pltpu.sync_copy(
            zeros_hbm.at[pl.ds(o, n)], acc.at[pl.ds(o, n)]))
    pltpu.sync_copy(ones_hbm, ones_v)
    plsc.subcore_barrier()
    base = wid * EPW
    for cols_hbm, acc in ((c1_hbm, acc0), (c2_hbm, acc1), (c3_hbm, acc2)):
        def body(g, carry, cols_hbm=cols_hbm, acc=acc):
            eb = pl.multiple_of(base + g * CH, 8)
            pltpu.sync_copy(cols_hbm.at[pl.ds(eb, CH)], idx_v)
            pltpu.sync_copy(ones_v, acc.at[idx_v], add=True)
            return carry
        lax.fori_loop(0, NCHUNK, body, 0)
    plsc.subcore_barrier()
    for i, acc in enumerate((acc0, acc1, acc2)):
        _striped(s, lambda o, n, i=i, acc=acc: pltpu.sync_copy(
            acc.at[pl.ds(o, n)], out_hbm.at[c, i, pl.ds(o, n)]))


_deg_call = pl.kernel(
    _deg_body,
    out_type=jax.ShapeDtypeStruct((NC, 3, N, DEG_W), jnp.float32),
    mesh=_MESH,
    compiler_params=_SC_PARAMS,
    scratch_types=[
        pltpu.VMEM((CH, DEG_W), jnp.float32),
        pltpu.VMEM((CH,), jnp.int32),
        pltpu.VMEM_SHARED((N, DEG_W), jnp.float32),
        pltpu.VMEM_SHARED((N, DEG_W), jnp.float32),
        pltpu.VMEM_SHARED((N, DEG_W), jnp.float32),
    ],
)


def _edge_body(y_hbm, rows_hbm, cols_hbm, zeros_hbm, out_hbm,
               row_v, col_v, data_v, acc_sh):
    c = lax.axis_index("c")
    s = lax.axis_index("s")
    wid = c * NS + s

    @pl.when(c == 0)
    def _():
        _striped(s, lambda o, n: pltpu.sync_copy(
            y_hbm.at[pl.ds(o, n)], acc_sh.at[pl.ds(o, n)]))

    @pl.when(c != 0)
    def _():
        _striped(s, lambda o, n: pltpu.sync_copy(
            zeros_hbm.at[pl.ds(o, n)], acc_sh.at[pl.ds(o, n)]))

    plsc.subcore_barrier()
    base = wid * EPW

    def body(g, carry):
        eb = pl.multiple_of(base + g * CH, 8)
        pltpu.sync_copy(rows_hbm.at[pl.ds(eb, CH)], row_v)
        pltpu.sync_copy(cols_hbm.at[pl.ds(eb, CH)], col_v)
        pltpu.sync_copy(y_hbm.at[row_v], data_v)
        pltpu.sync_copy(data_v, acc_sh.at[col_v], add=True)
        return carry

    lax.fori_loop(0, NCHUNK, body, 0)
    plsc.subcore_barrier()
    _striped(s, lambda o, n: pltpu.sync_copy(
        acc_sh.at[pl.ds(o, n)], out_hbm.at[c, pl.ds(o, n)]))


@functools.cache
def _edge_call(d):
    return pl.kernel(
        _edge_body,
        out_type=jax.ShapeDtypeStruct((NC, N, d), jnp.float32),
        mesh=_MESH,
        compiler_params=_SC_PARAMS,
        scratch_types=[
            pltpu.VMEM((CH,), jnp.int32),
            pltpu.VMEM((CH,), jnp.int32),
            pltpu.VMEM((CH, d), jnp.float32),
            pltpu.VMEM_SHARED((N, d), jnp.float32),
        ],
    )


def _first_body(x_ref, w_ref, d_ref, y_ref):
    dis = lax.rsqrt(d_ref[0] + d_ref[1] + 1.0)
    y_ref[...] = jnp.dot(x_ref[...], w_ref[...],
                         preferred_element_type=jnp.float32) * dis


def _mid_body(p_ref, d_ref, b_ref, w_ref, dn_ref, y_ref):
    dis = lax.rsqrt(d_ref[0] + d_ref[1] + 1.0)
    h = jnp.maximum((p_ref[0] + p_ref[1]) * dis + b_ref[...], 0.0)
    disn = lax.rsqrt(dn_ref[0] + dn_ref[1] + 1.0)
    y_ref[...] = jnp.dot(h, w_ref[...],
                         preferred_element_type=jnp.float32) * disn


def _final_body(p_ref, d_ref, b_ref, out_ref):
    dis = lax.rsqrt(d_ref[0] + d_ref[1] + 1.0)
    out_ref[...] = jnp.maximum((p_ref[0] + p_ref[1]) * dis + b_ref[...], 0.0)


def _tc(body, out_shape, *args):
    return pl.pallas_call(
        body, out_shape=jax.ShapeDtypeStruct(out_shape, jnp.float32))(*args)


def kernel(features, edge_indexes_1, edge_indexes_3, edge_indexes_9,
           W1, b1, W2, b2, W3, b3):
    rows1, cols1 = edge_indexes_1[0], edge_indexes_1[1]
    rows2, cols2 = edge_indexes_3[0], edge_indexes_3[1]
    rows3, cols3 = edge_indexes_9[0], edge_indexes_9[1]

    ones = jnp.ones((CH, DEG_W), jnp.float32)
    zeros64 = jnp.zeros((N, 64), jnp.float32)

    degp = _deg_call(cols1, cols2, cols3, ones, zeros64[:, :DEG_W])
    d1 = degp[:, 0, :, 0:1]
    d2 = degp[:, 1, :, 0:1]
    d3 = degp[:, 2, :, 0:1]

    y1 = _tc(_first_body, (N, 64), features, W1, d1)
    p1 = _edge_call(64)(y1, rows1, cols1, zeros64)
    y2 = _tc(_mid_body, (N, 32), p1, d1, b1, W2, d2)
    p2 = _edge_call(32)(y2, rows2, cols2, zeros64[:, :32])
    y3 = _tc(_mid_body, (N, 16), p2, d2, b2, W3, d3)
    p3 = _edge_call(16)(y3, rows3, cols3, zeros64[:, :16])
    h3 = _tc(_final_body, (N, 16), p3, d3, b3)
    return h3


# speedup vs baseline: 48.3841x; 2.7757x over previous
import functools

import jax
import jax.numpy as jnp
from jax import lax
from jax.experimental import pallas as pl
from jax.experimental.pallas import tpu as pltpu
from jax.experimental.pallas import tpu_sc as plsc

N = 10000
E = 320000
NC = 2
NS = 16
NW = NC * NS
EPW = E // NW
BLK = 200
NB = EPW // BLK
NPAIR = NB // 2
RPS = 624
TAIL = N - NS * RPS
DEG_W = 8

_MESH = plsc.VectorSubcoreMesh(core_axis_name="c", subcore_axis_name="s")
_SC_PARAMS = pltpu.CompilerParams(use_tc_tiling_on_sc=False)


def _striped(s, mk):
    mk(s * RPS, RPS)

    @pl.when(s == NS - 1)
    def _():
        mk(NS * RPS, TAIL)


def _blk(base, j):
    return pl.ds(pl.multiple_of(base + j * BLK, 8), BLK)


def _deg_body(c1_hbm, c2_hbm, c3_hbm, ones_hbm, zeros_hbm, out_hbm,
              ones_v, col_a, col_b, acc0, acc1, acc2, sem_ca, sem_cb):
    c = lax.axis_index("c")
    s = lax.axis_index("s")
    wid = c * NS + s
    ebase = wid * EPW
    for acc in (acc0, acc1, acc2):
        _striped(s, lambda o, n, acc=acc: pltpu.sync_copy(
            zeros_hbm.at[pl.ds(o, n)], acc.at[pl.ds(o, n)]))
    pltpu.sync_copy(ones_hbm, ones_v)
    plsc.subcore_barrier()

    for cols_hbm, acc in ((c1_hbm, acc0), (c2_hbm, acc1), (c3_hbm, acc2)):
        def start(j, buf, sem, cols_hbm=cols_hbm):
            pltpu.async_copy(cols_hbm.at[_blk(ebase, j)], buf, sem)

        def drain(j, buf, sem, cols_hbm=cols_hbm, acc=acc):
            pltpu.make_async_copy(
                cols_hbm.at[_blk(ebase, j)], buf, sem).wait()
            pltpu.sync_copy(ones_v, acc.at[buf], add=True)

        start(0, col_a, sem_ca)

        def pair(t, carry, start=start, drain=drain):
            j0 = 2 * t
            start(j0 + 1, col_b, sem_cb)
            drain(j0, col_a, sem_ca)

            @pl.when(t + 1 < NPAIR)
            def _():
                start(j0 + 2, col_a, sem_ca)

            drain(j0 + 1, col_b, sem_cb)
            return carry

        lax.fori_loop(0, NPAIR, pair, 0)
    plsc.subcore_barrier()
    for i, acc in enumerate((acc0, acc1, acc2)):
        _striped(s, lambda o, n, i=i, acc=acc: pltpu.sync_copy(
            acc.at[pl.ds(o, n)], out_hbm.at[c, i, pl.ds(o, n)]))


_deg_call = pl.kernel(
    _deg_body,
    out_type=jax.ShapeDtypeStruct((NC, 3, N, DEG_W), jnp.float32),
    mesh=_MESH,
    compiler_params=_SC_PARAMS,
    scratch_types=[
        pltpu.VMEM((BLK, DEG_W), jnp.float32),
        pltpu.VMEM((BLK,), jnp.int32),
        pltpu.VMEM((BLK,), jnp.int32),
        pltpu.VMEM_SHARED((N, DEG_W), jnp.float32),
        pltpu.VMEM_SHARED((N, DEG_W), jnp.float32),
        pltpu.VMEM_SHARED((N, DEG_W), jnp.float32),
        pltpu.SemaphoreType.DMA,
        pltpu.SemaphoreType.DMA,
    ],
)


def _edge_body(y_hbm, rows_hbm, cols_hbm, zeros_hbm, out_hbm,
               row_all, col_a, col_b, data_a, data_b, acc_sh,
               sem_ca, sem_cb, sem_ga, sem_gb):
    c = lax.axis_index("c")
    s = lax.axis_index("s")
    wid = c * NS + s
    ebase = wid * EPW
    pltpu.sync_copy(rows_hbm.at[pl.ds(ebase, EPW)], row_all)

    @pl.when(c == 0)
    def _():
        _striped(s, lambda o, n: pltpu.sync_copy(
            y_hbm.at[pl.ds(o, n)], acc_sh.at[pl.ds(o, n)]))

    @pl.when(c != 0)
    def _():
        _striped(s, lambda o, n: pltpu.sync_copy(
            zeros_hbm.at[pl.ds(o, n)], acc_sh.at[pl.ds(o, n)]))

    plsc.subcore_barrier()

    def start(j, colbuf, databuf, sem_c, sem_g):
        pltpu.async_copy(cols_hbm.at[_blk(ebase, j)], colbuf, sem_c)
        pltpu.async_copy(y_hbm.at[row_all.at[_blk(0, j)]], databuf, sem_g)

    def drain(j, colbuf, databuf, sem_c, sem_g):
        pltpu.make_async_copy(
            cols_hbm.at[_blk(ebase, j)], colbuf, sem_c).wait()
        pltpu.make_async_copy(
            y_hbm.at[row_all.at[_blk(0, j)]], databuf, sem_g).wait()
        pltpu.sync_copy(databuf, acc_sh.at[colbuf], add=True)

    start(0, col_a, data_a, sem_ca, sem_ga)

    def pair(t, carry):
        j0 = 2 * t
        start(j0 + 1, col_b, data_b, sem_cb, sem_gb)
        drain(j0, col_a, data_a, sem_ca, sem_ga)

        @pl.when(t + 1 < NPAIR)
        def _():
            start(j0 + 2, col_a, data_a, sem_ca, sem_ga)

        drain(j0 + 1, col_b, data_b, sem_cb, sem_gb)
        return carry

    lax.fori_loop(0, NPAIR, pair, 0)
    plsc.subcore_barrier()
    _striped(s, lambda o, n: pltpu.sync_copy(
        acc_sh.at[pl.ds(o, n)], out_hbm.at[c, pl.ds(o, n)]))


@functools.cache
def _edge_call(d):
    return pl.kernel(
        _edge_body,
        out_type=jax.ShapeDtypeStruct((NC, N, d), jnp.float32),
        mesh=_MESH,
        compiler_params=_SC_PARAMS,
        scratch_types=[
            pltpu.VMEM((EPW,), jnp.int32),
            pltpu.VMEM((BLK,), jnp.int32),
            pltpu.VMEM((BLK,), jnp.int32),
            pltpu.VMEM((BLK, d), jnp.float32),
            pltpu.VMEM((BLK, d), jnp.float32),
            pltpu.VMEM_SHARED((N, d), jnp.float32),
            pltpu.SemaphoreType.DMA,
            pltpu.SemaphoreType.DMA,
            pltpu.SemaphoreType.DMA,
            pltpu.SemaphoreType.DMA,
        ],
    )


def _first_body(x_ref, w_ref, d_ref, y_ref):
    dis = lax.rsqrt(d_ref[0] + d_ref[1] + 1.0)
    y_ref[...] = jnp.dot(x_ref[...], w_ref[...],
                         preferred_element_type=jnp.float32) * dis


def _mid_body(p_ref, d_ref, b_ref, w_ref, dn_ref, y_ref):
    dis = lax.rsqrt(d_ref[0] + d_ref[1] + 1.0)
    h = jnp.maximum((p_ref[0] + p_ref[1]) * dis + b_ref[...], 0.0)
    disn = lax.rsqrt(dn_ref[0] + dn_ref[1] + 1.0)
    y_ref[...] = jnp.dot(h, w_ref[...],
                         preferred_element_type=jnp.float32) * disn


def _final_body(p_ref, d_ref, b_ref, out_ref):
    dis = lax.rsqrt(d_ref[0] + d_ref[1] + 1.0)
    out_ref[...] = jnp.maximum((p_ref[0] + p_ref[1]) * dis + b_ref[...], 0.0)


def _tc(body, out_shape, *args):
    return pl.pallas_call(
        body, out_shape=jax.ShapeDtypeStruct(out_shape, jnp.float32))(*args)


def kernel(features, edge_indexes_1, edge_indexes_3, edge_indexes_9,
           W1, b1, W2, b2, W3, b3):
    rows1, cols1 = edge_indexes_1[0], edge_indexes_1[1]
    rows2, cols2 = edge_indexes_3[0], edge_indexes_3[1]
    rows3, cols3 = edge_indexes_9[0], edge_indexes_9[1]

    ones = jnp.ones((BLK, DEG_W), jnp.float32)
    zeros64 = jnp.zeros((N, 64), jnp.float32)

    degp = _deg_call(cols1, cols2, cols3, ones, zeros64[:, :DEG_W])
    d1 = degp[:, 0, :, 0:1]
    d2 = degp[:, 1, :, 0:1]
    d3 = degp[:, 2, :, 0:1]

    y1 = _tc(_first_body, (N, 64), features, W1, d1)
    p1 = _edge_call(64)(y1, rows1, cols1, zeros64)
    y2 = _tc(_mid_body, (N, 32), p1, d1, b1, W2, d2)
    p2 = _edge_call(32)(y2, rows2, cols2, zeros64[:, :32])
    y3 = _tc(_mid_body, (N, 16), p2, d2, b2, W3, d3)
    p3 = _edge_call(16)(y3, rows3, cols3, zeros64[:, :16])
    h3 = _tc(_final_body, (N, 16), p3, d3, b3)
    return h3


# speedup vs baseline: 55.6284x; 1.1497x over previous
import functools

import jax
import jax.numpy as jnp
from jax import lax
from jax.experimental import pallas as pl
from jax.experimental.pallas import tpu as pltpu
from jax.experimental.pallas import tpu_sc as plsc

N = 10000
E = 320000
NC = 2
NS = 16
NW = NC * NS
EPW = E // NW
RPS = 624
TAIL = N - NS * RPS
DEG_W = 8
DBLK = 1000
DNB = EPW // DBLK
_EDGE_BLK = {64: 400, 32: 1000, 16: 1000}

_MESH = plsc.VectorSubcoreMesh(core_axis_name="c", subcore_axis_name="s")
_SC_PARAMS = pltpu.CompilerParams(use_tc_tiling_on_sc=False)


def _striped(s, mk):
    mk(s * RPS, RPS)

    @pl.when(s == NS - 1)
    def _():
        mk(NS * RPS, TAIL)


def _deg_body(c1_hbm, c2_hbm, c3_hbm, ones_hbm, zeros_hbm, out_hbm,
              ones_v, col2d, acc0, acc1, acc2):
    c = lax.axis_index("c")
    s = lax.axis_index("s")
    wid = c * NS + s
    for acc in (acc0, acc1, acc2):
        _striped(s, lambda o, n, acc=acc: pltpu.sync_copy(
            zeros_hbm.at[pl.ds(o, n)], acc.at[pl.ds(o, n)]))
    pltpu.sync_copy(ones_hbm, ones_v)
    plsc.subcore_barrier()
    for cols_hbm, acc in ((c1_hbm, acc0), (c2_hbm, acc1), (c3_hbm, acc2)):
        pltpu.sync_copy(cols_hbm.at[wid], col2d)

        def body(j, carry, acc=acc):
            pltpu.sync_copy(ones_v, acc.at[col2d.at[j]], add=True)
            return carry

        lax.fori_loop(0, DNB, body, 0)
    plsc.subcore_barrier()
    for i, acc in enumerate((acc0, acc1, acc2)):
        _striped(s, lambda o, n, i=i, acc=acc: pltpu.sync_copy(
            acc.at[pl.ds(o, n)], out_hbm.at[c, i, pl.ds(o, n)]))


_deg_call = pl.kernel(
    _deg_body,
    out_type=jax.ShapeDtypeStruct((NC, 3, N, DEG_W), jnp.float32),
    mesh=_MESH,
    compiler_params=_SC_PARAMS,
    scratch_types=[
        pltpu.VMEM((DBLK, DEG_W), jnp.float32),
        pltpu.VMEM((DNB, DBLK), jnp.int32),
        pltpu.VMEM_SHARED((N, DEG_W), jnp.float32),
        pltpu.VMEM_SHARED((N, DEG_W), jnp.float32),
        pltpu.VMEM_SHARED((N, DEG_W), jnp.float32),
    ],
)


def _make_edge_body(d, blk, nb):
    def body_fn(y_hbm, rows_hbm, cols_hbm, zeros_hbm, out_hbm,
                row_all, col2d, data_a, data_b, acc_sh, sem_a, sem_b):
        c = lax.axis_index("c")
        s = lax.axis_index("s")
        wid = c * NS + s
        ebase = wid * EPW
        pltpu.sync_copy(rows_hbm.at[pl.ds(ebase, EPW)], row_all)
        pltpu.sync_copy(cols_hbm.at[wid], col2d)

        @pl.when(c == 0)
        def _():
            _striped(s, lambda o, n: pltpu.sync_copy(
                y_hbm.at[pl.ds(o, n)], acc_sh.at[pl.ds(o, n)]))

        @pl.when(c != 0)
        def _():
            _striped(s, lambda o, n: pltpu.sync_copy(
                zeros_hbm.at[pl.ds(o, n)], acc_sh.at[pl.ds(o, n)]))

        plsc.subcore_barrier()

        def g_src(j):
            return y_hbm.at[
                row_all.at[pl.ds(pl.multiple_of(j * blk, 8), blk)]]

        def start(j, buf, sem):
            pltpu.async_copy(g_src(j), buf, sem)

        def proc(j, buf, sem, obuf, osem):
            @pl.when(j + 1 < nb)
            def _():
                start(j + 1, obuf, osem)

            pltpu.make_async_copy(g_src(j), buf, sem).wait()
            pltpu.sync_copy(buf, acc_sh.at[col2d.at[j]], add=True)

        start(0, data_a, sem_a)

        def body(j, carry):
            @pl.when(lax.rem(j, 2) == 0)
            def _():
                proc(j, data_a, sem_a, data_b, sem_b)

            @pl.when(lax.rem(j, 2) == 1)
            def _():
                proc(j, data_b, sem_b, data_a, sem_a)

            return carry

        lax.fori_loop(0, nb, body, 0)
        plsc.subcore_barrier()
        _striped(s, lambda o, n: pltpu.sync_copy(
            acc_sh.at[pl.ds(o, n)], out_hbm.at[c, pl.ds(o, n)]))

    return body_fn


@functools.cache
def _edge_call(d):
    blk = _EDGE_BLK[d]
    nb = EPW // blk
    return pl.kernel(
        _make_edge_body(d, blk, nb),
        out_type=jax.ShapeDtypeStruct((NC, N, d), jnp.float32),
        mesh=_MESH,
        compiler_params=_SC_PARAMS,
        scratch_types=[
            pltpu.VMEM((EPW,), jnp.int32),
            pltpu.VMEM((nb, blk), jnp.int32),
            pltpu.VMEM((blk, d), jnp.float32),
            pltpu.VMEM((blk, d), jnp.float32),
            pltpu.VMEM_SHARED((N, d), jnp.float32),
            pltpu.SemaphoreType.DMA,
            pltpu.SemaphoreType.DMA,
        ],
    )


def _first_body(x_ref, w_ref, d_ref, y_ref):
    dis = lax.rsqrt(d_ref[0] + d_ref[1] + 1.0)
    y_ref[...] = jnp.dot(x_ref[...], w_ref[...],
                         preferred_element_type=jnp.float32) * dis


def _mid_body(p_ref, d_ref, b_ref, w_ref, dn_ref, y_ref):
    dis = lax.rsqrt(d_ref[0] + d_ref[1] + 1.0)
    h = jnp.maximum((p_ref[0] + p_ref[1]) * dis + b_ref[...], 0.0)
    disn = lax.rsqrt(dn_ref[0] + dn_ref[1] + 1.0)
    y_ref[...] = jnp.dot(h, w_ref[...],
                         preferred_element_type=jnp.float32) * disn


def _final_body(p_ref, d_ref, b_ref, out_ref):
    dis = lax.rsqrt(d_ref[0] + d_ref[1] + 1.0)
    out_ref[...] = jnp.maximum((p_ref[0] + p_ref[1]) * dis + b_ref[...], 0.0)


def _tc(body, out_shape, *args):
    return pl.pallas_call(
        body, out_shape=jax.ShapeDtypeStruct(out_shape, jnp.float32))(*args)


def kernel(features, edge_indexes_1, edge_indexes_3, edge_indexes_9,
           W1, b1, W2, b2, W3, b3):
    def cshape(d):
        blk = _EDGE_BLK[d]
        return (NW, EPW // blk, blk)

    rows1, cols1 = edge_indexes_1[0], edge_indexes_1[1]
    rows2, cols2 = edge_indexes_3[0], edge_indexes_3[1]
    rows3, cols3 = edge_indexes_9[0], edge_indexes_9[1]
    dcols1 = cols1.reshape(NW, DNB, DBLK)
    dcols2 = cols2.reshape(NW, DNB, DBLK)
    dcols3 = cols3.reshape(NW, DNB, DBLK)

    ones = jnp.ones((DBLK, DEG_W), jnp.float32)
    zeros64 = jnp.zeros((N, 64), jnp.float32)

    degp = _deg_call(dcols1, dcols2, dcols3, ones, zeros64[:, :DEG_W])
    d1 = degp[:, 0, :, 0:1]
    d2 = degp[:, 1, :, 0:1]
    d3 = degp[:, 2, :, 0:1]

    y1 = _tc(_first_body, (N, 64), features, W1, d1)
    p1 = _edge_call(64)(y1, rows1, cols1.reshape(cshape(64)), zeros64)
    y2 = _tc(_mid_body, (N, 32), p1, d1, b1, W2, d2)
    p2 = _edge_call(32)(y2, rows2, cols2.reshape(cshape(32)),
                        zeros64[:, :32])
    y3 = _tc(_mid_body, (N, 16), p2, d2, b2, W3, d3)
    p3 = _edge_call(16)(y3, rows3, cols3.reshape(cshape(16)),
                        zeros64[:, :16])
    h3 = _tc(_final_body, (N, 16), p3, d3, b3)
    return h3


# speedup vs baseline: 62.2457x; 1.1190x over previous
import functools

import jax
import jax.numpy as jnp
from jax import lax
from jax.experimental import pallas as pl
from jax.experimental.pallas import tpu as pltpu
from jax.experimental.pallas import tpu_sc as plsc

N = 10000
E = 320000
NC = 2
NS = 16
NW = NC * NS
EPW = E // NW
RPS = 624
TAIL = N - NS * RPS
DEG_W = 8
DBLK = 1000
DNB = EPW // DBLK
_EDGE_BLK = {64: 400, 32: 1000, 16: 1000}

_MESH = plsc.VectorSubcoreMesh(core_axis_name="c", subcore_axis_name="s")
_SC_PARAMS = pltpu.CompilerParams(use_tc_tiling_on_sc=False)


def _striped(s, mk):
    mk(s * RPS, RPS)

    @pl.when(s == NS - 1)
    def _():
        mk(NS * RPS, TAIL)


def _blk(base, j, blk):
    return pl.ds(pl.multiple_of(base + j * blk, 8), blk)


def _deg_body(e1_hbm, e2_hbm, e3_hbm, ones_hbm, zeros_hbm, out_hbm,
              ones_v, col_a, col_b, acc0, acc1, acc2, sem_a, sem_b):
    c = lax.axis_index("c")
    s = lax.axis_index("s")
    wid = c * NS + s
    ebase = wid * EPW
    for acc in (acc0, acc1, acc2):
        _striped(s, lambda o, n, acc=acc: pltpu.sync_copy(
            zeros_hbm.at[pl.ds(o, n)], acc.at[pl.ds(o, n)]))
    pltpu.sync_copy(ones_hbm, ones_v)
    plsc.subcore_barrier()

    for e_hbm, acc in ((e1_hbm, acc0), (e2_hbm, acc1), (e3_hbm, acc2)):
        def start(j, buf, sem, e_hbm=e_hbm):
            pltpu.async_copy(e_hbm.at[1, _blk(ebase, j, DBLK)], buf, sem)

        def proc(j, buf, sem, obuf, osem, e_hbm=e_hbm, acc=acc,
                 start=start):
            @pl.when(j + 1 < DNB)
            def _():
                start(j + 1, obuf, osem)

            pltpu.make_async_copy(
                e_hbm.at[1, _blk(ebase, j, DBLK)], buf, sem).wait()
            pltpu.sync_copy(ones_v, acc.at[buf], add=True)

        start(0, col_a, sem_a)

        def body(j, carry, proc=proc):
            @pl.when(lax.rem(j, 2) == 0)
            def _():
                proc(j, col_a, sem_a, col_b, sem_b)

            @pl.when(lax.rem(j, 2) == 1)
            def _():
                proc(j, col_b, sem_b, col_a, sem_a)

            return carry

        lax.fori_loop(0, DNB, body, 0)
    plsc.subcore_barrier()
    for i, acc in enumerate((acc0, acc1, acc2)):
        _striped(s, lambda o, n, i=i, acc=acc: pltpu.sync_copy(
            acc.at[pl.ds(o, n)], out_hbm.at[c, i, pl.ds(o, n)]))


_deg_call = pl.kernel(
    _deg_body,
    out_type=jax.ShapeDtypeStruct((NC, 3, N, DEG_W), jnp.float32),
    mesh=_MESH,
    compiler_params=_SC_PARAMS,
    scratch_types=[
        pltpu.VMEM((DBLK, DEG_W), jnp.float32),
        pltpu.VMEM((DBLK,), jnp.int32),
        pltpu.VMEM((DBLK,), jnp.int32),
        pltpu.VMEM_SHARED((N, DEG_W), jnp.float32),
        pltpu.VMEM_SHARED((N, DEG_W), jnp.float32),
        pltpu.VMEM_SHARED((N, DEG_W), jnp.float32),
        pltpu.SemaphoreType.DMA,
        pltpu.SemaphoreType.DMA,
    ],
)


def _make_edge_body(d, blk, nb):
    def body_fn(y_hbm, e_hbm, zeros_hbm, out_hbm,
                row_all, col_a, col_b, data_a, data_b, acc_sh,
                sem_ca, sem_cb, sem_ga, sem_gb):
        c = lax.axis_index("c")
        s = lax.axis_index("s")
        wid = c * NS + s
        ebase = wid * EPW
        pltpu.sync_copy(e_hbm.at[0, pl.ds(ebase, EPW)], row_all)

        @pl.when(c == 0)
        def _():
            _striped(s, lambda o, n: pltpu.sync_copy(
                y_hbm.at[pl.ds(o, n)], acc_sh.at[pl.ds(o, n)]))

        @pl.when(c != 0)
        def _():
            _striped(s, lambda o, n: pltpu.sync_copy(
                zeros_hbm.at[pl.ds(o, n)], acc_sh.at[pl.ds(o, n)]))

        plsc.subcore_barrier()

        def g_src(j):
            return y_hbm.at[row_all.at[_blk(0, j, blk)]]

        def start(j, cbuf, dbuf, sem_c, sem_g):
            pltpu.async_copy(e_hbm.at[1, _blk(ebase, j, blk)], cbuf, sem_c)
            pltpu.async_copy(g_src(j), dbuf, sem_g)

        def proc(j, cbuf, dbuf, sem_c, sem_g, ocbuf, odbuf, osem_c, osem_g):
            @pl.when(j + 1 < nb)
            def _():
                start(j + 1, ocbuf, odbuf, osem_c, osem_g)

            pltpu.make_async_copy(
                e_hbm.at[1, _blk(ebase, j, blk)], cbuf, sem_c).wait()
            pltpu.make_async_copy(g_src(j), dbuf, sem_g).wait()
            pltpu.sync_copy(dbuf, acc_sh.at[cbuf], add=True)

        start(0, col_a, data_a, sem_ca, sem_ga)

        def body(j, carry):
            @pl.when(lax.rem(j, 2) == 0)
            def _():
                proc(j, col_a, data_a, sem_ca, sem_ga,
                     col_b, data_b, sem_cb, sem_gb)

            @pl.when(lax.rem(j, 2) == 1)
            def _():
                proc(j, col_b, data_b, sem_cb, sem_gb,
                     col_a, data_a, sem_ca, sem_ga)

            return carry

        lax.fori_loop(0, nb, body, 0)
        plsc.subcore_barrier()
        _striped(s, lambda o, n: pltpu.sync_copy(
            acc_sh.at[pl.ds(o, n)], out_hbm.at[c, pl.ds(o, n)]))

    return body_fn


@functools.cache
def _edge_call(d):
    blk = _EDGE_BLK[d]
    nb = EPW // blk
    return pl.kernel(
        _make_edge_body(d, blk, nb),
        out_type=jax.ShapeDtypeStruct((NC, N, d), jnp.float32),
        mesh=_MESH,
        compiler_params=_SC_PARAMS,
        scratch_types=[
            pltpu.VMEM((EPW,), jnp.int32),
            pltpu.VMEM((blk,), jnp.int32),
            pltpu.VMEM((blk,), jnp.int32),
            pltpu.VMEM((blk, d), jnp.float32),
            pltpu.VMEM((blk, d), jnp.float32),
            pltpu.VMEM_SHARED((N, d), jnp.float32),
            pltpu.SemaphoreType.DMA,
            pltpu.SemaphoreType.DMA,
            pltpu.SemaphoreType.DMA,
            pltpu.SemaphoreType.DMA,
        ],
    )


def _dis(degp_ref, i):
    d8 = degp_ref[0, i] + degp_ref[1, i] + 1.0
    return lax.rsqrt(d8[:, 0:1])


def _first_body(x_ref, w_ref, degp_ref, y_ref):
    y_ref[...] = jnp.dot(x_ref[...], w_ref[...],
                         preferred_element_type=jnp.float32) * _dis(degp_ref, 0)


def _make_mid_body(i):
    def body(p_ref, degp_ref, b_ref, w_ref, y_ref):
        h = jnp.maximum(
            (p_ref[0] + p_ref[1]) * _dis(degp_ref, i) + b_ref[...], 0.0)
        y_ref[...] = jnp.dot(h, w_ref[...],
                             preferred_element_type=jnp.float32) * _dis(
                                 degp_ref, i + 1)
    return body


def _final_body(p_ref, degp_ref, b_ref, out_ref):
    out_ref[...] = jnp.maximum(
        (p_ref[0] + p_ref[1]) * _dis(degp_ref, 2) + b_ref[...], 0.0)


def _tc(body, out_shape, *args):
    return pl.pallas_call(
        body, out_shape=jax.ShapeDtypeStruct(out_shape, jnp.float32))(*args)


def kernel(features, edge_indexes_1, edge_indexes_3, edge_indexes_9,
           W1, b1, W2, b2, W3, b3):
    ones = jnp.ones((DBLK, DEG_W), jnp.float32)
    zeros64 = jnp.zeros((N, 64), jnp.float32)

    degp = _deg_call(edge_indexes_1, edge_indexes_3, edge_indexes_9,
                     ones, zeros64[:, :DEG_W])

    y1 = _tc(_first_body, (N, 64), features, W1, degp)
    p1 = _edge_call(64)(y1, edge_indexes_1, zeros64)
    y2 = _tc(_make_mid_body(0), (N, 32), p1, degp, b1, W2)
    p2 = _edge_call(32)(y2, edge_indexes_3, zeros64[:, :32])
    y3 = _tc(_make_mid_body(1), (N, 16), p2, degp, b2, W3)
    p3 = _edge_call(16)(y3, edge_indexes_9, zeros64[:, :16])
    h3 = _tc(_final_body, (N, 16), p3, degp, b3)
    return h3


# speedup vs baseline: 76.4939x; 1.2289x over previous
import functools

import jax
import jax.numpy as jnp
from jax import lax
from jax.experimental import pallas as pl
from jax.experimental.pallas import tpu as pltpu
from jax.experimental.pallas import tpu_sc as plsc

N = 10000
E = 320000
NC = 2
NS = 16
NW = NC * NS
EPW = E // NW
RPS = 624
TAIL = N - NS * RPS
DEG_W = 8
DBLK = 1000
DNB = EPW // DBLK
_EDGE_BLK = {64: 400, 32: 1000, 16: 1000}

_MESH = plsc.VectorSubcoreMesh(core_axis_name="c", subcore_axis_name="s")
_SC_PARAMS = pltpu.CompilerParams(use_tc_tiling_on_sc=False)


def _striped(s, mk):
    mk(s * RPS, RPS)

    @pl.when(s == NS - 1)
    def _():
        mk(NS * RPS, TAIL)


def _blk(base, j, blk):
    return pl.ds(pl.multiple_of(base + j * blk, 8), blk)


def _deg_body(e1_hbm, e2_hbm, e3_hbm, ones_hbm, zeros_hbm, out_hbm,
              ones_v, col_a, col_b, acc0, acc1, acc2, sem_a, sem_b):
    c = lax.axis_index("c")
    s = lax.axis_index("s")
    wid = c * NS + s
    ebase = wid * EPW
    for acc in (acc0, acc1, acc2):
        _striped(s, lambda o, n, acc=acc: pltpu.sync_copy(
            zeros_hbm.at[pl.ds(o, n)], acc.at[pl.ds(o, n)]))
    pltpu.sync_copy(ones_hbm, ones_v)
    plsc.subcore_barrier()

    for e_hbm, acc in ((e1_hbm, acc0), (e2_hbm, acc1), (e3_hbm, acc2)):
        def start(j, buf, sem, e_hbm=e_hbm):
            pltpu.async_copy(e_hbm.at[1, _blk(ebase, j, DBLK)], buf, sem)

        def proc(j, buf, sem, obuf, osem, e_hbm=e_hbm, acc=acc,
                 start=start):
            @pl.when(j + 1 < DNB)
            def _():
                start(j + 1, obuf, osem)

            pltpu.make_async_copy(
                e_hbm.at[1, _blk(ebase, j, DBLK)], buf, sem).wait()
            pltpu.sync_copy(ones_v, acc.at[buf], add=True)

        start(0, col_a, sem_a)

        def body(j, carry, proc=proc):
            @pl.when(lax.rem(j, 2) == 0)
            def _():
                proc(j, col_a, sem_a, col_b, sem_b)

            @pl.when(lax.rem(j, 2) == 1)
            def _():
                proc(j, col_b, sem_b, col_a, sem_a)

            return carry

        lax.fori_loop(0, DNB, body, 0)
    plsc.subcore_barrier()
    for cc in range(NC):
        @pl.when(c == cc)
        def _(cc=cc):
            for i, acc in enumerate((acc0, acc1, acc2)):
                co = 64 * cc + 16 * i
                _striped(s, lambda o, n, co=co, acc=acc: pltpu.sync_copy(
                    acc.at[pl.ds(o, n)],
                    out_hbm.at[pl.ds(o, n), pl.ds(co, DEG_W)]))


_deg_call = pl.kernel(
    _deg_body,
    out_type=jax.ShapeDtypeStruct((N, 128), jnp.float32),
    mesh=_MESH,
    compiler_params=_SC_PARAMS,
    scratch_types=[
        pltpu.VMEM((DBLK, DEG_W), jnp.float32),
        pltpu.VMEM((DBLK,), jnp.int32),
        pltpu.VMEM((DBLK,), jnp.int32),
        pltpu.VMEM_SHARED((N, DEG_W), jnp.float32),
        pltpu.VMEM_SHARED((N, DEG_W), jnp.float32),
        pltpu.VMEM_SHARED((N, DEG_W), jnp.float32),
        pltpu.SemaphoreType.DMA,
        pltpu.SemaphoreType.DMA,
    ],
)


def _make_edge_body(d, blk, nb):
    def body_fn(y_hbm, e_hbm, zeros_hbm, out_hbm,
                row_all, col_a, col_b, data_a, data_b, acc_sh,
                sem_ca, sem_cb, sem_ga, sem_gb):
        c = lax.axis_index("c")
        s = lax.axis_index("s")
        wid = c * NS + s
        ebase = wid * EPW
        pltpu.sync_copy(e_hbm.at[0, pl.ds(ebase, EPW)], row_all)

        @pl.when(c == 0)
        def _():
            _striped(s, lambda o, n: pltpu.sync_copy(
                y_hbm.at[pl.ds(o, n)], acc_sh.at[pl.ds(o, n)]))

        @pl.when(c != 0)
        def _():
            _striped(s, lambda o, n: pltpu.sync_copy(
                zeros_hbm.at[pl.ds(o, n)], acc_sh.at[pl.ds(o, n)]))

        plsc.subcore_barrier()

        def g_src(j):
            return y_hbm.at[row_all.at[_blk(0, j, blk)]]

        def start(j, cbuf, dbuf, sem_c, sem_g):
            pltpu.async_copy(e_hbm.at[1, _blk(ebase, j, blk)], cbuf, sem_c)
            pltpu.async_copy(g_src(j), dbuf, sem_g)

        def proc(j, cbuf, dbuf, sem_c, sem_g, ocbuf, odbuf, osem_c, osem_g):
            @pl.when(j + 1 < nb)
            def _():
                start(j + 1, ocbuf, odbuf, osem_c, osem_g)

            pltpu.make_async_copy(
                e_hbm.at[1, _blk(ebase, j, blk)], cbuf, sem_c).wait()
            pltpu.make_async_copy(g_src(j), dbuf, sem_g).wait()
            pltpu.sync_copy(dbuf, acc_sh.at[cbuf], add=True)

        start(0, col_a, data_a, sem_ca, sem_ga)

        def body(j, carry):
            @pl.when(lax.rem(j, 2) == 0)
            def _():
                proc(j, col_a, data_a, sem_ca, sem_ga,
                     col_b, data_b, sem_cb, sem_gb)

            @pl.when(lax.rem(j, 2) == 1)
            def _():
                proc(j, col_b, data_b, sem_cb, sem_gb,
                     col_a, data_a, sem_ca, sem_ga)

            return carry

        lax.fori_loop(0, nb, body, 0)
        plsc.subcore_barrier()
        for cc in range(NC):
            @pl.when(c == cc)
            def _(cc=cc):
                _striped(s, lambda o, n: pltpu.sync_copy(
                    acc_sh.at[pl.ds(o, n)],
                    out_hbm.at[pl.ds(o, n), pl.ds(64 * cc, d)]))

    return body_fn


@functools.cache
def _edge_call(d):
    blk = _EDGE_BLK[d]
    nb = EPW // blk
    return pl.kernel(
        _make_edge_body(d, blk, nb),
        out_type=jax.ShapeDtypeStruct((N, 128), jnp.float32),
        mesh=_MESH,
        compiler_params=_SC_PARAMS,
        scratch_types=[
            pltpu.VMEM((EPW,), jnp.int32),
            pltpu.VMEM((blk,), jnp.int32),
            pltpu.VMEM((blk,), jnp.int32),
            pltpu.VMEM((blk, d), jnp.float32),
            pltpu.VMEM((blk, d), jnp.float32),
            pltpu.VMEM_SHARED((N, d), jnp.float32),
            pltpu.SemaphoreType.DMA,
            pltpu.SemaphoreType.DMA,
            pltpu.SemaphoreType.DMA,
            pltpu.SemaphoreType.DMA,
        ],
    )


def _dis(deg_ref, i):
    d = (deg_ref[:, 16 * i:16 * i + 1]
         + deg_ref[:, 64 + 16 * i:64 + 16 * i + 1] + 1.0)
    return lax.rsqrt(d)


def _first_body(x_ref, w_ref, degp_ref, y_ref):
    y_ref[...] = jnp.dot(x_ref[...], w_ref[...],
                         preferred_element_type=jnp.float32) * _dis(degp_ref, 0)


def _make_mid_body(i):
    def body(p_ref, degp_ref, b_ref, w_ref, y_ref):
        d = w_ref.shape[0]
        h = jnp.maximum(
            (p_ref[:, 0:d] + p_ref[:, 64:64 + d]) * _dis(degp_ref, i)
            + b_ref[...], 0.0)
        y_ref[...] = jnp.dot(h, w_ref[...],
                             preferred_element_type=jnp.float32) * _dis(
                                 degp_ref, i + 1)
    return body


def _final_body(p_ref, degp_ref, b_ref, out_ref):
    out_ref[...] = jnp.maximum(
        (p_ref[:, 0:16] + p_ref[:, 64:80]) * _dis(degp_ref, 2)
        + b_ref[...], 0.0)


def _tc(body, out_shape, *args):
    return pl.pallas_call(
        body, out_shape=jax.ShapeDtypeStruct(out_shape, jnp.float32))(*args)


def kernel(features, edge_indexes_1, edge_indexes_3, edge_indexes_9,
           W1, b1, W2, b2, W3, b3):
    ones = jnp.ones((DBLK, DEG_W), jnp.float32)
    zeros64 = jnp.zeros((N, 64), jnp.float32)

    degp = _deg_call(edge_indexes_1, edge_indexes_3, edge_indexes_9,
                     ones, zeros64[:, :DEG_W])

    y1 = _tc(_first_body, (N, 64), features, W1, degp)
    p1 = _edge_call(64)(y1, edge_indexes_1, zeros64)
    y2 = _tc(_make_mid_body(0), (N, 32), p1, degp, b1, W2)
    p2 = _edge_call(32)(y2, edge_indexes_3, zeros64[:, :32])
    y3 = _tc(_make_mid_body(1), (N, 16), p2, degp, b2, W3)
    p3 = _edge_call(16)(y3, edge_indexes_9, zeros64[:, :16])
    h3 = _tc(_final_body, (N, 16), p3, degp, b3)
    return h3


# speedup vs baseline: 78.7139x; 1.0290x over previous
import functools

import jax
import jax.numpy as jnp
from jax import lax
from jax.experimental import pallas as pl
from jax.experimental.pallas import tpu as pltpu
from jax.experimental.pallas import tpu_sc as plsc

N = 10000
E = 320000
NC = 2
NS = 16
NW = NC * NS
EPW = E // NW
RPS = 624
TAIL = N - NS * RPS
DEG_W = 8
DBLK = 1000
DNB = EPW // DBLK
_EDGE_BLK = {64: 200, 32: 400, 16: 1000}

_MESH = plsc.VectorSubcoreMesh(core_axis_name="c", subcore_axis_name="s")
_SC_PARAMS = pltpu.CompilerParams(use_tc_tiling_on_sc=False)


def _striped(s, mk):
    mk(s * RPS, RPS)

    @pl.when(s == NS - 1)
    def _():
        mk(NS * RPS, TAIL)


def _blk(base, j, blk):
    return pl.ds(pl.multiple_of(base + j * blk, 8), blk)


def _deg_body(e1_hbm, e2_hbm, e3_hbm, ones_hbm, zeros_hbm, out_hbm,
              ones_v, col_a, col_b, acc0, acc1, acc2, sem_a, sem_b):
    c = lax.axis_index("c")
    s = lax.axis_index("s")
    wid = c * NS + s
    ebase = wid * EPW
    for acc in (acc0, acc1, acc2):
        _striped(s, lambda o, n, acc=acc: pltpu.sync_copy(
            zeros_hbm.at[pl.ds(o, n)], acc.at[pl.ds(o, n)]))
    pltpu.sync_copy(ones_hbm, ones_v)
    plsc.subcore_barrier()

    for e_hbm, acc in ((e1_hbm, acc0), (e2_hbm, acc1), (e3_hbm, acc2)):
        def start(j, buf, sem, e_hbm=e_hbm):
            pltpu.async_copy(e_hbm.at[1, _blk(ebase, j, DBLK)], buf, sem)

        def proc(j, buf, sem, obuf, osem, e_hbm=e_hbm, acc=acc,
                 start=start):
            @pl.when(j + 1 < DNB)
            def _():
                start(j + 1, obuf, osem)

            pltpu.make_async_copy(
                e_hbm.at[1, _blk(ebase, j, DBLK)], buf, sem).wait()
            pltpu.sync_copy(ones_v, acc.at[buf], add=True)

        start(0, col_a, sem_a)

        def body(j, carry, proc=proc):
            @pl.when(lax.rem(j, 2) == 0)
            def _():
                proc(j, col_a, sem_a, col_b, sem_b)

            @pl.when(lax.rem(j, 2) == 1)
            def _():
                proc(j, col_b, sem_b, col_a, sem_a)

            return carry

        lax.fori_loop(0, DNB, body, 0)
    plsc.subcore_barrier()
    for cc in range(NC):
        @pl.when(c == cc)
        def _(cc=cc):
            for i, acc in enumerate((acc0, acc1, acc2)):
                co = 64 * cc + 16 * i
                _striped(s, lambda o, n, co=co, acc=acc: pltpu.sync_copy(
                    acc.at[pl.ds(o, n)],
                    out_hbm.at[pl.ds(o, n), pl.ds(co, DEG_W)]))


_deg_call = pl.kernel(
    _deg_body,
    out_type=jax.ShapeDtypeStruct((N, 128), jnp.float32),
    mesh=_MESH,
    compiler_params=_SC_PARAMS,
    scratch_types=[
        pltpu.VMEM((DBLK, DEG_W), jnp.float32),
        pltpu.VMEM((DBLK,), jnp.int32),
        pltpu.VMEM((DBLK,), jnp.int32),
        pltpu.VMEM_SHARED((N, DEG_W), jnp.float32),
        pltpu.VMEM_SHARED((N, DEG_W), jnp.float32),
        pltpu.VMEM_SHARED((N, DEG_W), jnp.float32),
        pltpu.SemaphoreType.DMA,
        pltpu.SemaphoreType.DMA,
    ],
)


def _make_edge_body(d, blk, nb):
    R = 4

    def body_fn(y_hbm, e_hbm, zeros_hbm, out_hbm,
                row_all, col0, col1, col2, col3, dat0, dat1, dat2, dat3,
                acc_sh, sl0, sl1, sl2, sl3, ss0, ss1, ss2, ss3):
        cols = (col0, col1, col2, col3)
        dats = (dat0, dat1, dat2, dat3)
        sls = (sl0, sl1, sl2, sl3)
        sss = (ss0, ss1, ss2, ss3)
        c = lax.axis_index("c")
        s = lax.axis_index("s")
        wid = c * NS + s
        ebase = wid * EPW
        pltpu.sync_copy(e_hbm.at[0, pl.ds(ebase, EPW)], row_all)

        @pl.when(c == 0)
        def _():
            _striped(s, lambda o, n: pltpu.sync_copy(
                y_hbm.at[pl.ds(o, n)], acc_sh.at[pl.ds(o, n)]))

        @pl.when(c != 0)
        def _():
            _striped(s, lambda o, n: pltpu.sync_copy(
                zeros_hbm.at[pl.ds(o, n)], acc_sh.at[pl.ds(o, n)]))

        plsc.subcore_barrier()

        def col_desc(j, r):
            return (e_hbm.at[1, _blk(ebase, j, blk)], cols[r], sls[r])

        def g_desc(j, r):
            return (y_hbm.at[row_all.at[_blk(0, j, blk)]], dats[r], sls[r])

        def s_desc(r):
            return (dats[r], acc_sh.at[cols[r]], sss[r])

        def fire_loads(j, r):
            pltpu.async_copy(*col_desc(j, r))
            pltpu.async_copy(*g_desc(j, r))

        fire_loads(0, 0)
        fire_loads(1, 1)

        def step(j, r):
            r2 = (r + 2) % R

            @pl.when(j + 2 < nb)
            def _():
                fire_loads(j + 2, r2)

            pltpu.make_async_copy(*col_desc(j, r)).wait()
            pltpu.make_async_copy(*g_desc(j, r)).wait()
            pltpu.sync_copy(dats[r], acc_sh.at[cols[r]], add=True)

        def body(j, carry):
            for r in range(R):
                @pl.when(lax.rem(j, R) == r)
                def _(r=r):
                    step(j, r)

            return carry

        lax.fori_loop(0, nb, body, 0)
        plsc.subcore_barrier()
        for cc in range(NC):
            @pl.when(c == cc)
            def _(cc=cc):
                _striped(s, lambda o, n: pltpu.sync_copy(
                    acc_sh.at[pl.ds(o, n)],
                    out_hbm.at[pl.ds(o, n), pl.ds(64 * cc, d)]))

    return body_fn


@functools.cache
def _edge_call(d):
    blk = _EDGE_BLK[d]
    nb = EPW // blk
    return pl.kernel(
        _make_edge_body(d, blk, nb),
        out_type=jax.ShapeDtypeStruct((N, 128), jnp.float32),
        mesh=_MESH,
        compiler_params=_SC_PARAMS,
        scratch_types=(
            [pltpu.VMEM((EPW,), jnp.int32)]
            + [pltpu.VMEM((blk,), jnp.int32) for _ in range(4)]
            + [pltpu.VMEM((blk, d), jnp.float32) for _ in range(4)]
            + [pltpu.VMEM_SHARED((N, d), jnp.float32)]
            + [pltpu.SemaphoreType.DMA for _ in range(8)]
        ),
    )


def _dis(deg_ref, i):
    d = (deg_ref[:, 16 * i:16 * i + 1]
         + deg_ref[:, 64 + 16 * i:64 + 16 * i + 1] + 1.0)
    return lax.rsqrt(d)


def _first_body(x_ref, w_ref, degp_ref, y_ref):
    y_ref[...] = jnp.dot(x_ref[...], w_ref[...],
                         preferred_element_type=jnp.float32) * _dis(degp_ref, 0)


def _make_mid_body(i):
    def body(p_ref, degp_ref, b_ref, w_ref, y_ref):
        d = w_ref.shape[0]
        h = jnp.maximum(
            (p_ref[:, 0:d] + p_ref[:, 64:64 + d]) * _dis(degp_ref, i)
            + b_ref[...], 0.0)
        y_ref[...] = jnp.dot(h, w_ref[...],
                             preferred_element_type=jnp.float32) * _dis(
                                 degp_ref, i + 1)
    return body


def _final_body(p_ref, degp_ref, b_ref, out_ref):
    out_ref[...] = jnp.maximum(
        (p_ref[:, 0:16] + p_ref[:, 64:80]) * _dis(degp_ref, 2)
        + b_ref[...], 0.0)


def _tc(body, out_shape, *args):
    return pl.pallas_call(
        body, out_shape=jax.ShapeDtypeStruct(out_shape, jnp.float32))(*args)


def kernel(features, edge_indexes_1, edge_indexes_3, edge_indexes_9,
           W1, b1, W2, b2, W3, b3):
    ones = jnp.ones((DBLK, DEG_W), jnp.float32)
    zeros64 = jnp.zeros((N, 64), jnp.float32)

    degp = _deg_call(edge_indexes_1, edge_indexes_3, edge_indexes_9,
                     ones, zeros64[:, :DEG_W])

    y1 = _tc(_first_body, (N, 64), features, W1, degp)
    p1 = _edge_call(64)(y1, edge_indexes_1, zeros64)
    y2 = _tc(_make_mid_body(0), (N, 32), p1, degp, b1, W2)
    p2 = _edge_call(32)(y2, edge_indexes_3, zeros64[:, :32])
    y3 = _tc(_make_mid_body(1), (N, 16), p2, degp, b2, W3)
    p3 = _edge_call(16)(y3, edge_indexes_9, zeros64[:, :16])
    h3 = _tc(_final_body, (N, 16), p3, degp, b3)
    return h3
